# baseline (speedup 1.0000x reference)
import functools

import jax
import jax.numpy as jnp
from jax import lax
from jax.experimental import pallas as pl
from jax.experimental.pallas import tpu as pltpu
from jax.experimental.pallas import tpu_sc as plsc

B = 4096 * 200
D = 64
NC = 2
NS = 16
NW = NC * NS
B_PER_W = B // NW
C = 800
NCHUNK = B_PER_W // C

_mesh = plsc.VectorSubcoreMesh(core_axis_name="c", subcore_axis_name="s")


@functools.partial(
    pl.kernel,
    mesh=_mesh,
    out_type=jax.ShapeDtypeStruct((B, D), jnp.float32),
    scratch_types=[
        pltpu.VMEM((C,), jnp.int32),
        pltpu.VMEM((C, D), jnp.float32),
        pltpu.SemaphoreType.DMA,
    ],
    compiler_params=pltpu.CompilerParams(use_tc_tiling_on_sc=False),
)
def _emb_lookup(ids_hbm, table_hbm, out_hbm, idx_v, rows_v, sem):
    wid = lax.axis_index("s") * NC + lax.axis_index("c")
    base = wid * B_PER_W

    def chunk(i, carry):
        off = base + i * C
        pltpu.sync_copy(ids_hbm.at[pl.ds(off, C)], idx_v)
        pltpu.async_copy(table_hbm.at[idx_v], rows_v, sem).wait()
        pltpu.sync_copy(rows_v, out_hbm.at[pl.ds(off, C)])
        return carry

    lax.fori_loop(0, NCHUNK, chunk, 0)


def kernel(input_ids, table):
    flat = input_ids.reshape(-1)
    out = _emb_lookup(flat, table)
    return out.reshape(input_ids.shape[0], input_ids.shape[1], D)


# speedup vs baseline: 1.0247x; 1.0247x over previous
import functools

import jax
import jax.numpy as jnp
from jax import lax
from jax.experimental import pallas as pl
from jax.experimental.pallas import tpu as pltpu
from jax.experimental.pallas import tpu_sc as plsc

B = 4096 * 200
D = 64
NC = 2
NS = 16
NW = NC * NS
B_PER_W = B // NW
C = 800
NCHUNK = B_PER_W // C
NBUF = 2

_mesh = plsc.VectorSubcoreMesh(core_axis_name="c", subcore_axis_name="s")


@functools.partial(
    pl.kernel,
    mesh=_mesh,
    out_type=jax.ShapeDtypeStruct((B, D), jnp.float32),
    scratch_types=[
        [pltpu.VMEM((C,), jnp.int32) for _ in range(NBUF)],
        [pltpu.VMEM((C, D), jnp.float32) for _ in range(NBUF)],
        [pltpu.SemaphoreType.DMA for _ in range(NBUF)],
        [pltpu.SemaphoreType.DMA for _ in range(NBUF)],
        [pltpu.SemaphoreType.DMA for _ in range(NBUF)],
    ],
    compiler_params=pltpu.CompilerParams(use_tc_tiling_on_sc=False),
)
def _emb_lookup(ids_hbm, table_hbm, out_hbm, idx_v, rows_v,
                idx_sem, gat_sem, st_sem):
    wid = lax.axis_index("s") * NC + lax.axis_index("c")
    base = wid * B_PER_W

    def idx_load(i, b):
        pltpu.async_copy(ids_hbm.at[pl.ds(base + i * C, C)], idx_v[b],
                         idx_sem[b])

    for b in range(NBUF):
        idx_load(b, b)

    def step(g, carry):
        for b in range(NBUF):
            i = g * NBUF + b
            pltpu.make_async_copy(ids_hbm.at[pl.ds(base, C)], idx_v[b],
                                  idx_sem[b]).wait()
            @pl.when(g > 0)
            def _():
                pltpu.make_async_copy(rows_v[b],
                                      out_hbm.at[pl.ds(base, C)],
                                      st_sem[b]).wait()
            pltpu.async_copy(table_hbm.at[idx_v[b]], rows_v[b], gat_sem[b])
            pltpu.make_async_copy(table_hbm.at[idx_v[b]], rows_v[b],
                                  gat_sem[b]).wait()
            pltpu.async_copy(rows_v[b], out_hbm.at[pl.ds(base + i * C, C)],
                             st_sem[b])
            @pl.when(i + NBUF < NCHUNK)
            def _():
                idx_load(i + NBUF, b)
        return carry

    lax.fori_loop(0, NCHUNK // NBUF, step, 0)

    for b in range(NBUF):
        pltpu.make_async_copy(rows_v[b], out_hbm.at[pl.ds(base, C)],
                              st_sem[b]).wait()


def kernel(input_ids, table):
    flat = input_ids.reshape(-1)
    out = _emb_lookup(flat, table)
    return out.reshape(input_ids.shape[0], input_ids.shape[1], D)
